# baseline (speedup 1.0000x reference)
import functools

import jax
import jax.numpy as jnp
from jax import lax
from jax.experimental import pallas as pl
from jax.experimental.pallas import tpu as pltpu
from jax.experimental.pallas import tpu_sc as plsc

N_USERS = 1000000
N_ITEMS = 1000000
K = 32
BATCH = 16384

_NC = 2
_NS = 16
_NW = _NC * _NS
_BPW = BATCH // _NW
_L = 16
_NG = _BPW // _L


def _mf_body(users_hbm, items_hbm, pu_hbm, qi_hbm, out_hbm,
             idx_u, idx_i, u_rows, v_rows, out_v, sem):
    wid = lax.axis_index("s") * _NC + lax.axis_index("c")
    base = wid * _BPW

    pltpu.sync_copy(users_hbm.at[pl.ds(base, _BPW)], idx_u)
    pltpu.sync_copy(items_hbm.at[pl.ds(base, _BPW)], idx_i)

    cp_u = pltpu.async_copy(pu_hbm.at[idx_u], u_rows, sem)
    cp_v = pltpu.async_copy(qi_hbm.at[idx_i], v_rows, sem)
    cp_u.wait()
    cp_v.wait()

    lane = lax.iota(jnp.int32, _L)
    perms = [(lane ^ s).reshape(_L, 1) for s in (1, 2, 4, 8)]
    _dnums = lax.GatherDimensionNumbers(
        offset_dims=(), collapsed_slice_dims=(0,), start_index_map=(0,))

    def _perm(x, p):
        return lax.gather(x, p, _dnums, slice_sizes=(1,),
                          mode=lax.GatherScatterMode.PROMISE_IN_BOUNDS)

    def group(g, carry):
        base = g * _L
        acc = jnp.zeros((_L,), jnp.float32)
        for i in range(_L):
            b = base + i
            u0 = u_rows[b, pl.ds(0, _L)]
            v0 = v_rows[b, pl.ds(0, _L)]
            u1 = u_rows[b, pl.ds(_L, _L)]
            v1 = v_rows[b, pl.ds(_L, _L)]
            t = u0 * v0 + u1 * v1
            for p in perms:
                t = t + _perm(t, p)
            acc = jnp.where(lane == i, t, acc)
        out_v[pl.ds(base, _L)] = acc
        return carry

    lax.fori_loop(0, _NG, group, 0)

    pltpu.sync_copy(out_v, out_hbm.at[pl.ds(base, _BPW)])


@jax.jit
def _mf(users, items, pu_table, qi_table):
    mesh = plsc.VectorSubcoreMesh(core_axis_name="c", subcore_axis_name="s")
    f = functools.partial(
        pl.kernel,
        mesh=mesh,
        compiler_params=pltpu.CompilerParams(use_tc_tiling_on_sc=False),
        out_type=jax.ShapeDtypeStruct((BATCH,), jnp.float32),
        scratch_types=[
            pltpu.VMEM((_BPW,), jnp.int32),
            pltpu.VMEM((_BPW,), jnp.int32),
            pltpu.VMEM((_BPW, K), jnp.float32),
            pltpu.VMEM((_BPW, K), jnp.float32),
            pltpu.VMEM((_BPW,), jnp.float32),
            pltpu.SemaphoreType.DMA,
        ],
    )(_mf_body)
    return f(users, items, pu_table, qi_table)


def kernel(users, items, pu_table, qi_table):
    out = _mf(users.reshape(-1), items.reshape(-1), pu_table, qi_table)
    return out.reshape(-1, 1)
